# baseline (speedup 1.0000x reference)
import functools

import jax
import jax.numpy as jnp
from jax import lax
from jax.experimental import pallas as pl
from jax.experimental.pallas import tpu as pltpu
from jax.experimental.pallas import tpu_sc as plsc

N = 10000
E = 320000
H = 64
G = 512

NC = 2
NS = 16
NW = NC * NS

EC = 80
ECH = 125
EW = EC * ECH
NT = N
NPT = N // NS
ZR = 125

NP = 10240
PC = 80
PCH = 4
GPT = G // NS


def _mesh():
    return plsc.VectorSubcoreMesh(
        core_axis_name="c", subcore_axis_name="s", num_cores=NC, num_subcores=NS
    )


_SC_PARAMS = pltpu.CompilerParams(use_tc_tiling_on_sc=False)


def _zero_rows(ref, nrows, width):
    z = jnp.zeros((16,), jnp.float32)
    for r in range(nrows):
        for j in range(width // 16):
            ref[r, pl.ds(j * 16, 16)] = z


def _deg_body(edge_hbm, out_hbm, dst_v, ones_v, zb_v, acc_sh, sem):
    cid = lax.axis_index("c")
    sid = lax.axis_index("s")
    w = cid * NS + sid
    one = jnp.ones((16,), jnp.float32)
    for r in range(EC):
        ones_v[r, pl.ds(0, 16)] = one
    _zero_rows(zb_v, ZR, 16)
    base = sid * NPT
    for k in range(5):
        pltpu.sync_copy(zb_v, acc_sh.at[pl.ds(base + k * ZR, ZR)])
    plsc.subcore_barrier()
    pltpu.sync_copy(edge_hbm.at[1, pl.ds(w * EW, EW)], dst_v)

    def fire(i, carry):
        pltpu.async_copy(ones_v, acc_sh.at[dst_v.at[pl.ds(i * EC, EC)]], sem, add=True)
        return carry

    lax.fori_loop(0, ECH, fire, 0)

    def drain(i, carry):
        pltpu.make_async_copy(ones_v, acc_sh.at[dst_v.at[pl.ds(i * EC, EC)]], sem).wait()
        return carry

    lax.fori_loop(0, ECH, drain, 0)
    plsc.subcore_barrier()
    for k in range(5):
        pltpu.sync_copy(acc_sh.at[pl.ds(base + k * ZR, ZR)], zb_v)
        pltpu.sync_copy(zb_v, out_hbm.at[cid, pl.ds(base + k * ZR, ZR)])


_deg_call = functools.partial(
    pl.kernel,
    out_type=jax.ShapeDtypeStruct((NC, N, 16), jnp.float32),
    mesh=_mesh(),
    compiler_params=_SC_PARAMS,
    scratch_types=[
        pltpu.VMEM((EW,), jnp.int32),
        pltpu.VMEM((EC, 16), jnp.float32),
        pltpu.VMEM((ZR, 16), jnp.float32),
        pltpu.VMEM_SHARED((NT, 16), jnp.float32),
        pltpu.SemaphoreType.DMA,
    ],
)(_deg_body)


NB = 4


def _pass_body(m_hbm, edge_hbm, out_hbm, src_v, dst_v, r0, r1, r2, r3,
               zb_v, acc_sh, sg0, sg1, sg2, sg3, ss0, ss1, ss2, ss3):
    cid = lax.axis_index("c")
    sid = lax.axis_index("s")
    w = cid * NS + sid
    _zero_rows(zb_v, ZR, H)
    base = sid * NPT
    for k in range(5):
        pltpu.sync_copy(zb_v, acc_sh.at[pl.ds(base + k * ZR, ZR)])
    pltpu.sync_copy(edge_hbm.at[0, pl.ds(w * EW, EW)], src_v)
    pltpu.sync_copy(edge_hbm.at[1, pl.ds(w * EW, EW)], dst_v)
    plsc.subcore_barrier()

    rows = (r0, r1, r2, r3)
    sg = (sg0, sg1, sg2, sg3)
    ss = (ss0, ss1, ss2, ss3)

    def sidx(i):
        return src_v.at[pl.ds(i * EC, EC)]

    def didx(i):
        return dst_v.at[pl.ds(i * EC, EC)]

    def wait_g(i, b):
        pltpu.make_async_copy(m_hbm.at[sidx(i)], rows[b], sg[b]).wait()

    def wait_s(i, b):
        pltpu.make_async_copy(rows[b], acc_sh.at[didx(i)], ss[b]).wait()

    def start_g(i, b):
        pltpu.async_copy(m_hbm.at[sidx(i)], rows[b], sg[b])

    def start_s(i, b):
        pltpu.async_copy(rows[b], acc_sh.at[didx(i)], ss[b], add=True)

    def sched(i):
        b = i % NB
        wait_g(i, b)
        start_s(i, b)
        if i >= 2:
            wait_s(i - 2, (i - 2) % NB)
        if i + 2 < ECH:
            start_g(i + 2, (i + 2) % NB)

    start_g(0, 0)
    start_g(1, 1)
    sched(0)
    sched(1)
    NMAIN = (ECH - 2 - 3) // 4

    def body(g, carry):
        for k in range(4):
            i = 4 * g + 2 + k
            b = (2 + k) % 4
            wait_g(i, b)
            start_s(i, b)
            wait_s(i - 2, (b + 2) % 4)
            start_g(i + 2, (b + 2) % 4)
        return carry

    lax.fori_loop(0, NMAIN, body, 0)
    for i in range(2 + 4 * NMAIN, ECH):
        sched(i)
    for i in (ECH - 2, ECH - 1):
        wait_s(i, i % 4)
    plsc.subcore_barrier()
    for k in range(5):
        pltpu.sync_copy(acc_sh.at[pl.ds(base + k * ZR, ZR)], zb_v)
        pltpu.sync_copy(zb_v, out_hbm.at[cid, pl.ds(base + k * ZR, ZR)])


_pass_call = functools.partial(
    pl.kernel,
    out_type=jax.ShapeDtypeStruct((NC, N, H), jnp.float32),
    mesh=_mesh(),
    compiler_params=_SC_PARAMS,
    scratch_types=[
        pltpu.VMEM((EW,), jnp.int32),
        pltpu.VMEM((EW,), jnp.int32),
        pltpu.VMEM((EC, H), jnp.float32),
        pltpu.VMEM((EC, H), jnp.float32),
        pltpu.VMEM((EC, H), jnp.float32),
        pltpu.VMEM((EC, H), jnp.float32),
        pltpu.VMEM((ZR, H), jnp.float32),
        pltpu.VMEM_SHARED((NT, H), jnp.float32),
        pltpu.SemaphoreType.DMA,
        pltpu.SemaphoreType.DMA,
        pltpu.SemaphoreType.DMA,
        pltpu.SemaphoreType.DMA,
        pltpu.SemaphoreType.DMA,
        pltpu.SemaphoreType.DMA,
        pltpu.SemaphoreType.DMA,
        pltpu.SemaphoreType.DMA,
    ],
)(_pass_body)


def _pool_body(h_hbm, b_hbm, out_hbm, bidx_v, rows_v, zb_v, acc_sh):
    cid = lax.axis_index("c")
    sid = lax.axis_index("s")
    w = cid * NS + sid
    _zero_rows(zb_v, GPT, H)
    pltpu.sync_copy(zb_v, acc_sh.at[pl.ds(sid * GPT, GPT)])
    plsc.subcore_barrier()
    pltpu.sync_copy(b_hbm.at[pl.ds(w * PCH, PCH)], bidx_v)
    for k in range(PCH):
        pltpu.sync_copy(h_hbm.at[pl.ds(w * (PCH * PC) + k * PC, PC)], rows_v)
        pltpu.sync_copy(rows_v, acc_sh.at[bidx_v.at[k]], add=True)
    plsc.subcore_barrier()
    pltpu.sync_copy(acc_sh.at[pl.ds(sid * GPT, GPT)], zb_v)
    pltpu.sync_copy(zb_v, out_hbm.at[cid, pl.ds(sid * GPT, GPT)])


_pool_call = functools.partial(
    pl.kernel,
    out_type=jax.ShapeDtypeStruct((NC, G, H), jnp.float32),
    mesh=_mesh(),
    compiler_params=_SC_PARAMS,
    scratch_types=[
        pltpu.VMEM((PCH, PC), jnp.int32),
        pltpu.VMEM((PC, H), jnp.float32),
        pltpu.VMEM((GPT, H), jnp.float32),
        pltpu.VMEM_SHARED((G, H), jnp.float32),
    ],
)(_pool_body)


def _enc_body(x_ref, We_ref, be_ref, ga_ref, bt_ref, W1_ref, degp_ref,
              m1_ref, dinv_ref):
    h = jnp.dot(x_ref[...], We_ref[...], preferred_element_type=jnp.float32)
    h = h + be_ref[...][None, :]
    mean = jnp.mean(h, axis=0, keepdims=True)
    var = jnp.mean((h - mean) ** 2, axis=0, keepdims=True)
    hn = ga_ref[...][None, :] * (h - mean) * lax.rsqrt(var + 1e-5) + bt_ref[...][None, :]
    deg = degp_ref[0, :, 0:1] + degp_ref[1, :, 0:1] + 1.0
    dinv = lax.rsqrt(deg)
    m1_ref[...] = jnp.dot(hn, W1_ref[...], preferred_element_type=jnp.float32) * dinv
    dinv_ref[...] = dinv


def _enc_call(x, We, be, ga, bt, W1, degp):
    return pl.pallas_call(
        _enc_body,
        out_shape=[
            jax.ShapeDtypeStruct((N, H), jnp.float32),
            jax.ShapeDtypeStruct((N, 1), jnp.float32),
        ],
    )(x, We, be, ga, bt, W1, degp)


def _layer_body(aggp_ref, mprev_ref, dinv_ref, b_ref, W_ref, out_ref):
    dinv = dinv_ref[...]
    h = dinv * (aggp_ref[0] + aggp_ref[1] + mprev_ref[...]) + b_ref[...][None, :]
    h = jnp.maximum(h, 0.0)
    out_ref[...] = jnp.dot(h, W_ref[...], preferred_element_type=jnp.float32) * dinv


_LBS = 2000


def _layer_call(aggp, mprev, dinv, b, Wnext):
    return pl.pallas_call(
        _layer_body,
        grid=(N // _LBS,),
        in_specs=[
            pl.BlockSpec((NC, _LBS, H), lambda i: (0, i, 0)),
            pl.BlockSpec((_LBS, H), lambda i: (i, 0)),
            pl.BlockSpec((_LBS, 1), lambda i: (i, 0)),
            pl.BlockSpec((H,), lambda i: (0,)),
            pl.BlockSpec((H, H), lambda i: (0, 0)),
        ],
        out_specs=pl.BlockSpec((_LBS, H), lambda i: (i, 0)),
        out_shape=jax.ShapeDtypeStruct((N, H), jnp.float32),
    )(aggp, mprev, dinv, b, Wnext)


def _layer3_body(aggp_ref, mprev_ref, dinv_ref, b_ref, out_ref):
    h = dinv_ref[...] * (aggp_ref[0] + aggp_ref[1] + mprev_ref[...]) + b_ref[...][None, :]
    out_ref[...] = jnp.concatenate(
        [h, jnp.zeros((NP - N, H), jnp.float32)], axis=0
    )


def _layer3_call(aggp, mprev, dinv, b):
    return pl.pallas_call(
        _layer3_body,
        out_shape=jax.ShapeDtypeStruct((NP, H), jnp.float32),
    )(aggp, mprev, dinv, b)


def _head_body(pp_ref, Wl1_ref, bl1_ref, Wc_ref, bc_ref, out_ref):
    pooled = pp_ref[0] + pp_ref[1]
    h = jnp.maximum(
        jnp.dot(pooled, Wl1_ref[...], preferred_element_type=jnp.float32)
        + bl1_ref[...][None, :], 0.0)
    out_ref[...] = (
        jnp.dot(h, Wc_ref[...], preferred_element_type=jnp.float32)
        + bc_ref[...][None, :]
    )


def _head_call(pp, Wl1, bl1, Wc, bc):
    return pl.pallas_call(
        _head_body,
        out_shape=jax.ShapeDtypeStruct((G, Wc.shape[1]), jnp.float32),
    )(pp, Wl1, bl1, Wc, bc)


def kernel(x, edge_index, batch, W_enc, b_enc, gamma, beta, W1, b1, W2, b2,
           W3, b3, Wl1, bl1, Wclf, bclf):
    degp = _deg_call(edge_index)
    m1, dinv = _enc_call(x, W_enc, b_enc, gamma, beta, W1, degp)

    agg1 = _pass_call(m1, edge_index)
    m2 = _layer_call(agg1, m1, dinv, b1, W2)
    agg2 = _pass_call(m2, edge_index)
    m3 = _layer_call(agg2, m2, dinv, b2, W3)
    agg3 = _pass_call(m3, edge_index)
    h3 = _layer3_call(agg3, m3, dinv, b3)

    batch2 = jnp.pad(batch, (0, NP - N)).reshape(NP // PC, PC)
    poolp = _pool_call(h3, batch2)
    return _head_call(poolp, Wl1, bl1, Wclf, bclf)


# speedup vs baseline: 1.0024x; 1.0024x over previous
import functools

import jax
import jax.numpy as jnp
from jax import lax
from jax.experimental import pallas as pl
from jax.experimental.pallas import tpu as pltpu
from jax.experimental.pallas import tpu_sc as plsc

N = 10000
E = 320000
H = 64
G = 512

NC = 2
NS = 16
NW = NC * NS

EC = 80
ECH = 125
EW = EC * ECH
NT = N
NPT = N // NS
ZR = 125

NP = 10240
PC = 80
PCH = 4
GPT = G // NS


def _mesh():
    return plsc.VectorSubcoreMesh(
        core_axis_name="c", subcore_axis_name="s", num_cores=NC, num_subcores=NS
    )


_SC_PARAMS = pltpu.CompilerParams(use_tc_tiling_on_sc=False)


def _zero_rows(ref, nrows, width):
    z = jnp.zeros((16,), jnp.float32)
    for r in range(nrows):
        for j in range(width // 16):
            ref[r, pl.ds(j * 16, 16)] = z


def _deg_body(edge_hbm, out_hbm, dst_v, ones_v, zb_v, acc_sh, sem):
    cid = lax.axis_index("c")
    sid = lax.axis_index("s")
    w = cid * NS + sid
    one = jnp.ones((16,), jnp.float32)
    for r in range(EC):
        ones_v[r, pl.ds(0, 16)] = one
    _zero_rows(zb_v, ZR, 16)
    base = sid * NPT
    for k in range(5):
        pltpu.sync_copy(zb_v, acc_sh.at[pl.ds(base + k * ZR, ZR)])
    plsc.subcore_barrier()
    pltpu.sync_copy(edge_hbm.at[1, pl.ds(w * EW, EW)], dst_v)

    def fire(i, carry):
        pltpu.async_copy(ones_v, acc_sh.at[dst_v.at[pl.ds(i * EC, EC)]], sem, add=True)
        return carry

    lax.fori_loop(0, ECH, fire, 0)

    def drain(i, carry):
        pltpu.make_async_copy(ones_v, acc_sh.at[dst_v.at[pl.ds(i * EC, EC)]], sem).wait()
        return carry

    lax.fori_loop(0, ECH, drain, 0)
    plsc.subcore_barrier()
    for k in range(5):
        pltpu.sync_copy(acc_sh.at[pl.ds(base + k * ZR, ZR)], zb_v)
        pltpu.sync_copy(zb_v, out_hbm.at[cid, pl.ds(base + k * ZR, ZR)])


_deg_call = functools.partial(
    pl.kernel,
    out_type=jax.ShapeDtypeStruct((NC, N, 16), jnp.float32),
    mesh=_mesh(),
    compiler_params=_SC_PARAMS,
    scratch_types=[
        pltpu.VMEM((EW,), jnp.int32),
        pltpu.VMEM((EC, 16), jnp.float32),
        pltpu.VMEM((ZR, 16), jnp.float32),
        pltpu.VMEM_SHARED((NT, 16), jnp.float32),
        pltpu.SemaphoreType.DMA,
    ],
)(_deg_body)


NB = 4


def _pass_body(m_hbm, edge_hbm, out_hbm, src_v, dst_v, r0, r1, r2, r3,
               zb_v, acc_sh, sg0, sg1, sg2, sg3, ss0, ss1, ss2, ss3):
    cid = lax.axis_index("c")
    sid = lax.axis_index("s")
    w = cid * NS + sid
    _zero_rows(zb_v, ZR, H)
    base = sid * NPT
    for k in range(5):
        pltpu.sync_copy(zb_v, acc_sh.at[pl.ds(base + k * ZR, ZR)])
    pltpu.sync_copy(edge_hbm.at[0, pl.ds(w * EW, EW)], src_v)
    pltpu.sync_copy(edge_hbm.at[1, pl.ds(w * EW, EW)], dst_v)
    plsc.subcore_barrier()

    rows = (r0, r1, r2, r3)
    sg = (sg0, sg1, sg2, sg3)
    ss = (ss0, ss1, ss2, ss3)

    def sidx(i):
        return src_v.at[pl.ds(i * EC, EC)]

    def didx(i):
        return dst_v.at[pl.ds(i * EC, EC)]

    def wait_g(i, b):
        pltpu.make_async_copy(m_hbm.at[sidx(i)], rows[b], sg[b]).wait()

    def wait_s(i, b):
        pltpu.make_async_copy(rows[b], acc_sh.at[didx(i)], ss[b]).wait()

    def start_g(i, b):
        pltpu.async_copy(m_hbm.at[sidx(i)], rows[b], sg[b])

    def start_s(i, b):
        pltpu.async_copy(rows[b], acc_sh.at[didx(i)], ss[b], add=True)

    def sched(i):
        b = i % NB
        wait_g(i, b)
        start_s(i, b)
        if i >= 2:
            wait_s(i - 2, (i - 2) % NB)
        if i + 2 < ECH:
            start_g(i + 2, (i + 2) % NB)

    start_g(0, 0)
    start_g(1, 1)
    sched(0)
    sched(1)
    NMAIN = (ECH - 2 - 3) // 4

    def body(g, carry):
        for k in range(4):
            i = 4 * g + 2 + k
            b = (2 + k) % 4
            wait_g(i, b)
            start_s(i, b)
            wait_s(i - 2, (b + 2) % 4)
            start_g(i + 2, (b + 2) % 4)
        return carry

    lax.fori_loop(0, NMAIN, body, 0)
    for i in range(2 + 4 * NMAIN, ECH):
        sched(i)
    for i in (ECH - 2, ECH - 1):
        wait_s(i, i % 4)
    plsc.subcore_barrier()
    for k in range(5):
        pltpu.sync_copy(acc_sh.at[pl.ds(base + k * ZR, ZR)], zb_v)
        pltpu.sync_copy(zb_v, out_hbm.at[cid, pl.ds(base + k * ZR, ZR)])


_pass_call = functools.partial(
    pl.kernel,
    out_type=jax.ShapeDtypeStruct((NC, N, H), jnp.float32),
    mesh=_mesh(),
    compiler_params=_SC_PARAMS,
    scratch_types=[
        pltpu.VMEM((EW,), jnp.int32),
        pltpu.VMEM((EW,), jnp.int32),
        pltpu.VMEM((EC, H), jnp.float32),
        pltpu.VMEM((EC, H), jnp.float32),
        pltpu.VMEM((EC, H), jnp.float32),
        pltpu.VMEM((EC, H), jnp.float32),
        pltpu.VMEM((ZR, H), jnp.float32),
        pltpu.VMEM_SHARED((NT, H), jnp.float32),
        pltpu.SemaphoreType.DMA,
        pltpu.SemaphoreType.DMA,
        pltpu.SemaphoreType.DMA,
        pltpu.SemaphoreType.DMA,
        pltpu.SemaphoreType.DMA,
        pltpu.SemaphoreType.DMA,
        pltpu.SemaphoreType.DMA,
        pltpu.SemaphoreType.DMA,
    ],
)(_pass_body)


def _pool_body(h_hbm, b_hbm, out_hbm, bidx_v, rows_v, zb_v, acc_sh):
    cid = lax.axis_index("c")
    sid = lax.axis_index("s")
    w = cid * NS + sid
    _zero_rows(zb_v, GPT, H)
    pltpu.sync_copy(zb_v, acc_sh.at[pl.ds(sid * GPT, GPT)])
    plsc.subcore_barrier()
    pltpu.sync_copy(b_hbm.at[pl.ds(w * PCH, PCH)], bidx_v)
    for k in range(PCH):
        pltpu.sync_copy(h_hbm.at[pl.ds(w * (PCH * PC) + k * PC, PC)], rows_v)
        pltpu.sync_copy(rows_v, acc_sh.at[bidx_v.at[k]], add=True)
    plsc.subcore_barrier()
    pltpu.sync_copy(acc_sh.at[pl.ds(sid * GPT, GPT)], zb_v)
    pltpu.sync_copy(zb_v, out_hbm.at[cid, pl.ds(sid * GPT, GPT)])


_pool_call = functools.partial(
    pl.kernel,
    out_type=jax.ShapeDtypeStruct((NC, G, H), jnp.float32),
    mesh=_mesh(),
    compiler_params=_SC_PARAMS,
    scratch_types=[
        pltpu.VMEM((PCH, PC), jnp.int32),
        pltpu.VMEM((PC, H), jnp.float32),
        pltpu.VMEM((GPT, H), jnp.float32),
        pltpu.VMEM_SHARED((G, H), jnp.float32),
    ],
)(_pool_body)


def _enc_bn_body(x_ref, We_ref, be_ref, ga_ref, bt_ref, hn_ref):
    h = jnp.dot(x_ref[...], We_ref[...], preferred_element_type=jnp.float32)
    h = h + be_ref[...][None, :]
    mean = jnp.mean(h, axis=0, keepdims=True)
    var = jnp.mean((h - mean) ** 2, axis=0, keepdims=True)
    hn_ref[...] = (
        ga_ref[...][None, :] * (h - mean) * lax.rsqrt(var + 1e-5)
        + bt_ref[...][None, :]
    )


def _enc_scale_body(hn_ref, W1_ref, degp_ref, m1_ref, dinv_ref):
    deg = degp_ref[0, :, 0:1] + degp_ref[1, :, 0:1] + 1.0
    dinv = lax.rsqrt(deg)
    m1_ref[...] = jnp.dot(hn_ref[...], W1_ref[...], preferred_element_type=jnp.float32) * dinv
    dinv_ref[...] = dinv


def _enc_call(x, We, be, ga, bt, W1, degp):
    hn = pl.pallas_call(
        _enc_bn_body,
        out_shape=jax.ShapeDtypeStruct((N, H), jnp.float32),
    )(x, We, be, ga, bt)
    return pl.pallas_call(
        _enc_scale_body,
        out_shape=[
            jax.ShapeDtypeStruct((N, H), jnp.float32),
            jax.ShapeDtypeStruct((N, 1), jnp.float32),
        ],
    )(hn, W1, degp)


def _layer_body(aggp_ref, mprev_ref, dinv_ref, b_ref, W_ref, out_ref):
    dinv = dinv_ref[...]
    h = dinv * (aggp_ref[0] + aggp_ref[1] + mprev_ref[...]) + b_ref[...][None, :]
    h = jnp.maximum(h, 0.0)
    out_ref[...] = jnp.dot(h, W_ref[...], preferred_element_type=jnp.float32) * dinv


_LBS = 2000


def _layer_call(aggp, mprev, dinv, b, Wnext):
    return pl.pallas_call(
        _layer_body,
        grid=(N // _LBS,),
        in_specs=[
            pl.BlockSpec((NC, _LBS, H), lambda i: (0, i, 0)),
            pl.BlockSpec((_LBS, H), lambda i: (i, 0)),
            pl.BlockSpec((_LBS, 1), lambda i: (i, 0)),
            pl.BlockSpec((H,), lambda i: (0,)),
            pl.BlockSpec((H, H), lambda i: (0, 0)),
        ],
        out_specs=pl.BlockSpec((_LBS, H), lambda i: (i, 0)),
        out_shape=jax.ShapeDtypeStruct((N, H), jnp.float32),
    )(aggp, mprev, dinv, b, Wnext)


def _layer3_body(aggp_ref, mprev_ref, dinv_ref, b_ref, out_ref):
    h = dinv_ref[...] * (aggp_ref[0] + aggp_ref[1] + mprev_ref[...]) + b_ref[...][None, :]
    out_ref[...] = jnp.concatenate(
        [h, jnp.zeros((NP - N, H), jnp.float32)], axis=0
    )


def _layer3_call(aggp, mprev, dinv, b):
    return pl.pallas_call(
        _layer3_body,
        out_shape=jax.ShapeDtypeStruct((NP, H), jnp.float32),
    )(aggp, mprev, dinv, b)


def _head_body(pp_ref, Wl1_ref, bl1_ref, Wc_ref, bc_ref, out_ref):
    pooled = pp_ref[0] + pp_ref[1]
    h = jnp.maximum(
        jnp.dot(pooled, Wl1_ref[...], preferred_element_type=jnp.float32)
        + bl1_ref[...][None, :], 0.0)
    out_ref[...] = (
        jnp.dot(h, Wc_ref[...], preferred_element_type=jnp.float32)
        + bc_ref[...][None, :]
    )


def _head_call(pp, Wl1, bl1, Wc, bc):
    return pl.pallas_call(
        _head_body,
        out_shape=jax.ShapeDtypeStruct((G, Wc.shape[1]), jnp.float32),
    )(pp, Wl1, bl1, Wc, bc)


def kernel(x, edge_index, batch, W_enc, b_enc, gamma, beta, W1, b1, W2, b2,
           W3, b3, Wl1, bl1, Wclf, bclf):
    degp = _deg_call(edge_index)
    m1, dinv = _enc_call(x, W_enc, b_enc, gamma, beta, W1, degp)

    agg1 = _pass_call(m1, edge_index)
    m2 = _layer_call(agg1, m1, dinv, b1, W2)
    agg2 = _pass_call(m2, edge_index)
    m3 = _layer_call(agg2, m2, dinv, b2, W3)
    agg3 = _pass_call(m3, edge_index)
    h3 = _layer3_call(agg3, m3, dinv, b3)

    batch2 = jnp.pad(batch, (0, NP - N)).reshape(NP // PC, PC)
    poolp = _pool_call(h3, batch2)
    return _head_call(poolp, Wl1, bl1, Wclf, bclf)
